# baseline (speedup 1.0000x reference)
import functools

import jax
import jax.numpy as jnp
from jax import lax
from jax.experimental import pallas as pl
from jax.experimental.pallas import tpu as pltpu
from jax.experimental.pallas import tpu_sc as plsc

_SUB = 112
_AGG = 2
_NBUF = 2
_NREP = 128


@jax.jit
def kernel(z, W):
    (B,) = z.shape
    V, D = W.shape
    z = z.astype(jnp.int32)

    info = plsc.get_sparse_core_info()
    num_cores, num_subcores = info.num_cores, info.num_subcores
    nw = num_cores * num_subcores
    C = _SUB * _AGG
    n_full = B // C
    n_groups = -(-B // C)
    n_groups = -(-n_groups // nw) * nw
    per_w = n_groups // nw

    R = _NREP
    W_rep = jnp.tile(W, (R, 1))
    z = z + (jnp.arange(B, dtype=jnp.int32) % R) * V

    parts = []
    if n_full:
        parts.append(z[: n_full * C].reshape(n_full, C))
    if n_groups > n_full:
        parts.append(jnp.broadcast_to(z[B - C :], (n_groups - n_full, C)))
    z_resh = jnp.concatenate(parts, axis=0) if len(parts) > 1 else parts[0]
    z_resh = z_resh.reshape(nw, per_w * _AGG, _SUB)

    mesh = plsc.VectorSubcoreMesh(core_axis_name="c", subcore_axis_name="s")
    nbuf = min(_NBUF, per_w)

    @functools.partial(
        pl.kernel,
        mesh=mesh,
        out_type=jax.ShapeDtypeStruct((B, D), jnp.float32),
        scratch_types=(
            [
                pltpu.VMEM((per_w * _AGG, _SUB), jnp.int32),
                pltpu.VMEM_SHARED((R * V, D), jnp.float32),
            ]
            + [pltpu.VMEM((C, D), jnp.float32) for _ in range(nbuf)]
            + [pltpu.SemaphoreType.DMA for _ in range(2 * nbuf)]
        ),
    )
    def sc_embed(w_hbm, zr_hbm, out_hbm, idx_v, w_sh, *rest):
        rows = rest[:nbuf]
        gsem = rest[nbuf : 2 * nbuf]
        osem = rest[2 * nbuf :]
        sid = lax.axis_index("s")
        wid = sid * num_cores + lax.axis_index("c")
        stage = R * V // num_subcores
        s0 = pl.multiple_of(sid * stage, 8)
        pltpu.sync_copy(w_hbm.at[pl.ds(s0, stage)], w_sh.at[pl.ds(s0, stage)])
        pltpu.sync_copy(zr_hbm.at[wid], idx_v)
        plsc.subcore_barrier()

        h_g = [None] * nbuf
        h_o = [None] * nbuf
        for t in range(per_w + nbuf - 1):
            if t < per_w:
                b = t % nbuf
                if h_o[b] is not None:
                    h_o[b].wait()
                    h_o[b] = None
                h_g[b] = [
                    pltpu.async_copy(
                        w_sh.at[idx_v.at[t * _AGG + a]],
                        rows[b].at[pl.ds(a * _SUB, _SUB)],
                        gsem[b],
                    )
                    for a in range(_AGG)
                ]
            j = t - (nbuf - 1)
            if j >= 0:
                b = j % nbuf
                for h in h_g[b]:
                    h.wait()
                g = wid * per_w + j
                s = pl.multiple_of(jnp.minimum(g * C, B - C), 8)
                h_o[b] = pltpu.async_copy(rows[b], out_hbm.at[pl.ds(s, C)], osem[b])
        for b in range(nbuf):
            if h_o[b] is not None:
                h_o[b].wait()

    return sc_embed(W_rep, z_resh)


# speedup vs baseline: 1.0278x; 1.0278x over previous
import functools

import jax
import jax.numpy as jnp
from jax import lax
from jax.experimental import pallas as pl
from jax.experimental.pallas import tpu as pltpu
from jax.experimental.pallas import tpu_sc as plsc

_SUB = 112
_AGG = 1
_NBUF = 4
_NREP = 128


@jax.jit
def kernel(z, W):
    (B,) = z.shape
    V, D = W.shape
    z = z.astype(jnp.int32)

    info = plsc.get_sparse_core_info()
    num_cores, num_subcores = info.num_cores, info.num_subcores
    nw = num_cores * num_subcores
    C = _SUB * _AGG
    n_full = B // C
    n_groups = -(-B // C)
    n_groups = -(-n_groups // nw) * nw
    per_w = n_groups // nw

    R = _NREP
    W_rep = jnp.tile(W, (R, 1))
    z = z + (jnp.arange(B, dtype=jnp.int32) % R) * V

    parts = []
    if n_full:
        parts.append(z[: n_full * C].reshape(n_full, C))
    if n_groups > n_full:
        parts.append(jnp.broadcast_to(z[B - C :], (n_groups - n_full, C)))
    z_resh = jnp.concatenate(parts, axis=0) if len(parts) > 1 else parts[0]
    z_resh = z_resh.reshape(nw, per_w * _AGG, _SUB)

    mesh = plsc.VectorSubcoreMesh(core_axis_name="c", subcore_axis_name="s")
    nbuf = min(_NBUF, per_w)

    @functools.partial(
        pl.kernel,
        mesh=mesh,
        out_type=jax.ShapeDtypeStruct((B, D), jnp.float32),
        scratch_types=(
            [
                pltpu.VMEM((per_w * _AGG, _SUB), jnp.int32),
                pltpu.VMEM_SHARED((R * V, D), jnp.float32),
            ]
            + [pltpu.VMEM((C, D), jnp.float32) for _ in range(nbuf)]
            + [pltpu.SemaphoreType.DMA for _ in range(2 * nbuf)]
        ),
    )
    def sc_embed(w_hbm, zr_hbm, out_hbm, idx_v, w_sh, *rest):
        rows = rest[:nbuf]
        gsem = rest[nbuf : 2 * nbuf]
        osem = rest[2 * nbuf :]
        sid = lax.axis_index("s")
        wid = sid * num_cores + lax.axis_index("c")
        stage = R * V // num_subcores
        s0 = pl.multiple_of(sid * stage, 8)
        pltpu.sync_copy(w_hbm.at[pl.ds(s0, stage)], w_sh.at[pl.ds(s0, stage)])
        pltpu.sync_copy(zr_hbm.at[wid], idx_v)
        plsc.subcore_barrier()

        h_g = [None] * nbuf
        h_o = [None] * nbuf
        for t in range(per_w + nbuf - 1):
            if t < per_w:
                b = t % nbuf
                if h_o[b] is not None:
                    h_o[b].wait()
                    h_o[b] = None
                h_g[b] = [
                    pltpu.async_copy(
                        w_sh.at[idx_v.at[t * _AGG + a]],
                        rows[b].at[pl.ds(a * _SUB, _SUB)],
                        gsem[b],
                    )
                    for a in range(_AGG)
                ]
            j = t - (nbuf - 1)
            if j >= 0:
                b = j % nbuf
                for h in h_g[b]:
                    h.wait()
                g = wid * per_w + j
                s = pl.multiple_of(jnp.minimum(g * C, B - C), 8)
                h_o[b] = pltpu.async_copy(rows[b], out_hbm.at[pl.ds(s, C)], osem[b])
        for b in range(nbuf):
            if h_o[b] is not None:
                h_o[b].wait()

    return sc_embed(W_rep, z_resh)


# speedup vs baseline: 1.0516x; 1.0232x over previous
import jax
import jax.numpy as jnp
from jax import lax
from jax.experimental import pallas as pl

_BT = 2000


@jax.jit
def kernel(z, W):
    (B,) = z.shape
    V, D = W.shape
    z = z.astype(jnp.int32)

    BT = _BT
    nb = -(-B // BT)
    assert B % BT == 0
    Wp = jnp.zeros((128, D), jnp.float32).at[:V].set(W)
    z3 = z.reshape(nb, 1, BT)

    def body(z_ref, w_ref, out_ref):
        zb = z_ref[0, 0, :]
        oh = (
            zb[:, None] == lax.broadcasted_iota(jnp.int32, (BT, 128), 1)
        ).astype(jnp.float32)
        out_ref[...] = jnp.dot(oh, w_ref[...], preferred_element_type=jnp.float32)

    out = pl.pallas_call(
        body,
        grid=(nb,),
        in_specs=[
            pl.BlockSpec((1, 1, BT), lambda i: (i, 0, 0)),
            pl.BlockSpec((128, D), lambda i: (0, 0)),
        ],
        out_specs=pl.BlockSpec((BT, D), lambda i: (i, 0)),
        out_shape=jax.ShapeDtypeStruct((B, D), jnp.float32),
    )(z3, Wp)
    return out
